# baseline (speedup 1.0000x reference)
import functools

import jax
import jax.numpy as jnp
from jax import lax
from jax.experimental import pallas as pl
from jax.experimental.pallas import tpu as pltpu
from jax.experimental.pallas import tpu_sc as plsc

T = 2048
D = 1024
E = 64
K = 2
H = 512
C = 96
SHARED_H = 1024
ALPHA = 0.5
EPS = 1e-05
NPAIR = T * K
NSLOT = E * C
RT = 256
NW = 32


def _gelu(x):
    return 0.5 * x * (1.0 + lax.erf(x * 0.7071067811865476))


def _pack(a):
    n, w2 = a.shape
    return lax.bitcast_convert_type(a.reshape(n, w2 // 2, 2), jnp.float32)


def _unpack(a):
    n, w = a.shape
    return lax.bitcast_convert_type(a, jnp.bfloat16).reshape(n, 2 * w)


def _lnrt_body(x_ref, lns_ref, lnb_ref, rw_ref, rb_ref, h_ref, hb_ref, meta_ref):
    x = x_ref[...]
    mu = jnp.mean(x, axis=1, keepdims=True)
    xc = x - mu
    var = jnp.mean(xc * xc, axis=1, keepdims=True)
    h = xc * lax.rsqrt(var + EPS) * lns_ref[...] + lnb_ref[...]
    h_ref[...] = h
    hb_ref[...] = h.astype(jnp.bfloat16)
    logits = jnp.dot(h, rw_ref[...], preferred_element_type=jnp.float32) + rb_ref[...]
    col = lax.broadcasted_iota(jnp.int32, logits.shape, 1)
    v0 = jnp.max(logits, axis=1, keepdims=True)
    i0 = jnp.min(jnp.where(logits == v0, col, E), axis=1, keepdims=True)
    l2 = jnp.where(col == i0, -jnp.inf, logits)
    v1 = jnp.max(l2, axis=1, keepdims=True)
    i1 = jnp.min(jnp.where(l2 == v1, col, E), axis=1, keepdims=True)
    e1 = jnp.exp(v1 - v0)
    g0 = 1.0 / (1.0 + e1)
    g1 = e1 * g0
    meta_ref[...] = jnp.concatenate(
        [i0.astype(jnp.float32), i1.astype(jnp.float32), g0, g1], axis=1)


def _lnrt(x, lns, lnb, rw, rb):
    return pl.pallas_call(
        _lnrt_body,
        grid=(T // RT,),
        in_specs=[
            pl.BlockSpec((RT, D), lambda i: (i, 0)),
            pl.BlockSpec((1, D), lambda i: (0, 0)),
            pl.BlockSpec((1, D), lambda i: (0, 0)),
            pl.BlockSpec((D, E), lambda i: (0, 0)),
            pl.BlockSpec((1, E), lambda i: (0, 0)),
        ],
        out_specs=[
            pl.BlockSpec((RT, D), lambda i: (i, 0)),
            pl.BlockSpec((RT, D), lambda i: (i, 0)),
            pl.BlockSpec((RT, 4), lambda i: (i, 0)),
        ],
        out_shape=[
            jax.ShapeDtypeStruct((T, D), jnp.float32),
            jax.ShapeDtypeStruct((T, D), jnp.bfloat16),
            jax.ShapeDtypeStruct((T, 4), jnp.float32),
        ],
    )(x, lns, lnb, rw, rb)


def _pos_body(meta_ref, sidx_ref, slotcat_ref, keep_ref):
    e0 = meta_ref[:, 0:1].astype(jnp.int32)
    e1 = meta_ref[:, 1:2].astype(jnp.int32)
    iota = lax.broadcasted_iota(jnp.int32, (T, E), 1)
    oh0 = (e0 == iota).astype(jnp.int32)
    oh1 = (e1 == iota).astype(jnp.int32)
    both = oh0 + oh1
    s = both
    sh = 1
    while sh < T:
        top = jnp.zeros((sh, E), jnp.int32)
        s = s + jnp.concatenate([top, s[:-sh, :]], axis=0)
        sh *= 2
    s = s - both
    pos0 = jnp.sum(s * oh0, axis=1, keepdims=True)
    pos1 = jnp.sum(s * oh1, axis=1, keepdims=True)
    keep0 = pos0 < C
    keep1 = pos1 < C
    slot0 = e0 * C + jnp.minimum(pos0, C - 1)
    slot1 = e1 * C + jnp.minimum(pos1, C - 1)
    sidx_ref[...] = jnp.concatenate(
        [jnp.where(keep0, slot0, NSLOT), jnp.where(keep1, slot1, NSLOT)], axis=1)
    slotcat_ref[...] = jnp.concatenate([slot0, slot1], axis=0)
    keep_ref[...] = jnp.concatenate(
        [keep0.astype(jnp.float32), keep1.astype(jnp.float32)], axis=1)


def _pos(meta):
    return pl.pallas_call(
        _pos_body,
        out_shape=[
            jax.ShapeDtypeStruct((T, K), jnp.int32),
            jax.ShapeDtypeStruct((NPAIR, 1), jnp.int32),
            jax.ShapeDtypeStruct((T, K), jnp.float32),
        ],
    )(meta)


@functools.cache
def _sc_mesh():
    return plsc.VectorSubcoreMesh(core_axis_name="c", subcore_axis_name="s")


PW = D // 2


@functools.cache
def _make_dispatch(chunk):
    bw = NPAIR // NW
    nch = bw // chunk

    @functools.partial(
        pl.kernel,
        mesh=_sc_mesh(),
        out_type=jax.ShapeDtypeStruct((NSLOT + 8, PW), jnp.float32),
        scratch_types=[
            pltpu.VMEM((nch, chunk), jnp.int32),
            pltpu.VMEM((nch, chunk), jnp.int32),
            pltpu.VMEM((chunk, PW), jnp.float32),
            pltpu.SemaphoreType.DMA,
        ],
    )
    def k(h_hbm, tsrc_hbm, sidx_hbm, buf_hbm, tidx_v, sidx_v, rows_v, sem):
        wid = lax.axis_index("s") * 2 + lax.axis_index("c")
        pltpu.sync_copy(tsrc_hbm.at[wid], tidx_v)
        pltpu.sync_copy(sidx_hbm.at[wid], sidx_v)

        def body(c, carry):
            pltpu.async_copy(h_hbm.at[tidx_v.at[c]], rows_v, sem).wait()
            pltpu.async_copy(rows_v, buf_hbm.at[sidx_v.at[c]], sem).wait()
            return carry

        lax.fori_loop(0, nch, body, 0)

    return k


@functools.cache
def _make_gather(nrows_out, chunk):
    bw = nrows_out // NW
    nch = bw // chunk

    @functools.partial(
        pl.kernel,
        mesh=_sc_mesh(),
        out_type=jax.ShapeDtypeStruct((nrows_out, PW), jnp.float32),
        scratch_types=[
            pltpu.VMEM((nch, chunk), jnp.int32),
            pltpu.VMEM((chunk, PW), jnp.float32),
            pltpu.SemaphoreType.DMA,
        ],
    )
    def k(table_hbm, idx_hbm, out_hbm, idx_v, rows_v, sem):
        wid = lax.axis_index("s") * 2 + lax.axis_index("c")
        base = wid * bw
        pltpu.sync_copy(idx_hbm.at[wid], idx_v)

        def body(c, carry):
            off = pl.multiple_of(base + c * chunk, 8)
            pltpu.async_copy(table_hbm.at[idx_v.at[c]], rows_v, sem).wait()
            pltpu.sync_copy(rows_v, out_hbm.at[pl.ds(off, chunk)])
            return carry

        lax.fori_loop(0, nch, body, 0)

    return k


def _experts_body(buf_ref, w1_ref, b1_ref, w2_ref, b2_ref, out_ref):
    x = buf_ref[...].astype(jnp.float32)
    a = jnp.dot(x, w1_ref[0], preferred_element_type=jnp.float32) + b1_ref[0]
    a = _gelu(a)
    res = jnp.dot(a, w2_ref[0], preferred_element_type=jnp.float32) + b2_ref[0]
    out_ref[...] = res.astype(jnp.bfloat16)


def _experts(buf, w1, b1, w2, b2):
    return pl.pallas_call(
        _experts_body,
        grid=(E,),
        in_specs=[
            pl.BlockSpec((C, D), lambda e: (e, 0)),
            pl.BlockSpec((1, D, H), lambda e: (e, 0, 0)),
            pl.BlockSpec((1, 1, H), lambda e: (e, 0, 0)),
            pl.BlockSpec((1, H, D), lambda e: (e, 0, 0)),
            pl.BlockSpec((1, 1, D), lambda e: (e, 0, 0)),
        ],
        out_specs=pl.BlockSpec((C, D), lambda e: (e, 0)),
        out_shape=jax.ShapeDtypeStruct((NSLOT, D), jnp.bfloat16),
    )(buf, w1, b1, w2, b2)


def _final_body(x_ref, h_ref, ta_ref, tb_ref, meta_ref, keep_ref,
                w1_ref, b1_ref, w2_ref, b2_ref, out_ref):
    h = h_ref[...]
    a = _gelu(jnp.dot(h, w1_ref[...], preferred_element_type=jnp.float32) + b1_ref[...])
    sh = x_ref[...] + jnp.dot(a, w2_ref[...], preferred_element_type=jnp.float32) + b2_ref[...]
    gk0 = meta_ref[:, 2:3] * keep_ref[:, 0:1]
    gk1 = meta_ref[:, 3:4] * keep_ref[:, 1:2]
    out_ref[...] = sh + ALPHA * (gk0 * ta_ref[...].astype(jnp.float32)
                                 + gk1 * tb_ref[...].astype(jnp.float32))


def _final(x, h, tokrows, meta, keep2, w1, b1, w2, b2):
    return pl.pallas_call(
        _final_body,
        grid=(T // RT,),
        in_specs=[
            pl.BlockSpec((RT, D), lambda i: (i, 0)),
            pl.BlockSpec((RT, D), lambda i: (i, 0)),
            pl.BlockSpec((RT, D), lambda i: (i, 0)),
            pl.BlockSpec((RT, D), lambda i: (i + T // RT, 0)),
            pl.BlockSpec((RT, 4), lambda i: (i, 0)),
            pl.BlockSpec((RT, K), lambda i: (i, 0)),
            pl.BlockSpec((D, SHARED_H), lambda i: (0, 0)),
            pl.BlockSpec((1, SHARED_H), lambda i: (0, 0)),
            pl.BlockSpec((SHARED_H, D), lambda i: (0, 0)),
            pl.BlockSpec((1, D), lambda i: (0, 0)),
        ],
        out_specs=pl.BlockSpec((RT, D), lambda i: (i, 0)),
        out_shape=jax.ShapeDtypeStruct((T, D), jnp.float32),
    )(x, h, tokrows, tokrows, meta, keep2, w1, b1, w2, b2)


def kernel(hidden_states, ln_scale, ln_bias, shared_W1, shared_b1, shared_W2,
           shared_b2, router_W, router_b, expert_W1, expert_b1, expert_W2,
           expert_b2):
    x = hidden_states.reshape(T, D)
    h, hb, meta = _lnrt(x, ln_scale.reshape(1, D), ln_bias.reshape(1, D),
                        router_W, router_b.reshape(1, E))
    sidx2, slotcat, keep2 = _pos(meta)
    chunk = 64
    tsrc = jnp.repeat(jnp.arange(T, dtype=jnp.int32), K).reshape(NW, -1, chunk)
    buf = _make_dispatch(chunk)(_pack(hb), tsrc, sidx2.reshape(NW, -1, chunk))
    eo = _experts(_unpack(buf), expert_W1, expert_b1.reshape(E, 1, H),
                  expert_W2, expert_b2.reshape(E, 1, D))
    tokrows = _unpack(
        _make_gather(NPAIR, chunk)(_pack(eo), slotcat.reshape(NW, -1, chunk)))
    out = _final(x, h, tokrows, meta, keep2, shared_W1,
                 shared_b1.reshape(1, SHARED_H), shared_W2,
                 shared_b2.reshape(1, D))
    return out.reshape(1, T, D)


# speedup vs baseline: 3.2954x; 3.2954x over previous
import functools

import jax
import jax.numpy as jnp
from jax import lax
from jax.experimental import pallas as pl
from jax.experimental.pallas import tpu as pltpu
from jax.experimental.pallas import tpu_sc as plsc

T = 2048
D = 1024
E = 64
K = 2
H = 512
C = 96
SHARED_H = 1024
ALPHA = 0.5
EPS = 1e-05
NPAIR = T * K
NSLOT = E * C
RT = 256
NW = 32


def _gelu(x):
    return 0.5 * x * (1.0 + lax.erf(x * 0.7071067811865476))


def _pack_words(lo, hi):
    lo16 = lax.bitcast_convert_type(lo.astype(jnp.bfloat16),
                                    jnp.uint16).astype(jnp.uint32)
    hi16 = lax.bitcast_convert_type(hi.astype(jnp.bfloat16),
                                    jnp.uint16).astype(jnp.uint32)
    return lax.bitcast_convert_type(
        jnp.left_shift(hi16, jnp.uint32(16)) | lo16, jnp.float32)


def _unpack_lo(w):
    u = lax.bitcast_convert_type(w, jnp.uint32)
    return lax.bitcast_convert_type(
        jnp.left_shift(u, jnp.uint32(16)), jnp.float32)


def _unpack_hi(w):
    u = lax.bitcast_convert_type(w, jnp.uint32)
    return lax.bitcast_convert_type(u & jnp.uint32(0xFFFF0000), jnp.float32)


def _lnrt_body(x_ref, lns_ref, lnb_ref, rw_ref, rb_ref, h_ref, hb_ref, meta_ref):
    x = x_ref[...]
    mu = jnp.mean(x, axis=1, keepdims=True)
    xc = x - mu
    var = jnp.mean(xc * xc, axis=1, keepdims=True)
    h = xc * lax.rsqrt(var + EPS) * lns_ref[...] + lnb_ref[...]
    h_ref[...] = h
    hb_ref[...] = _pack_words(h[:, :D // 2], h[:, D // 2:])
    logits = jnp.dot(h, rw_ref[...], preferred_element_type=jnp.float32) + rb_ref[...]
    col = lax.broadcasted_iota(jnp.int32, logits.shape, 1)
    v0 = jnp.max(logits, axis=1, keepdims=True)
    i0 = jnp.min(jnp.where(logits == v0, col, E), axis=1, keepdims=True)
    l2 = jnp.where(col == i0, -jnp.inf, logits)
    v1 = jnp.max(l2, axis=1, keepdims=True)
    i1 = jnp.min(jnp.where(l2 == v1, col, E), axis=1, keepdims=True)
    e1 = jnp.exp(v1 - v0)
    g0 = 1.0 / (1.0 + e1)
    g1 = e1 * g0
    meta_ref[...] = jnp.concatenate(
        [i0.astype(jnp.float32), i1.astype(jnp.float32), g0, g1], axis=1)


def _lnrt(x, lns, lnb, rw, rb):
    return pl.pallas_call(
        _lnrt_body,
        grid=(T // RT,),
        in_specs=[
            pl.BlockSpec((RT, D), lambda i: (i, 0)),
            pl.BlockSpec((1, D), lambda i: (0, 0)),
            pl.BlockSpec((1, D), lambda i: (0, 0)),
            pl.BlockSpec((D, E), lambda i: (0, 0)),
            pl.BlockSpec((1, E), lambda i: (0, 0)),
        ],
        out_specs=[
            pl.BlockSpec((RT, D), lambda i: (i, 0)),
            pl.BlockSpec((RT, D // 2), lambda i: (i, 0)),
            pl.BlockSpec((RT, 4), lambda i: (i, 0)),
        ],
        out_shape=[
            jax.ShapeDtypeStruct((T, D), jnp.float32),
            jax.ShapeDtypeStruct((T, D // 2), jnp.float32),
            jax.ShapeDtypeStruct((T, 4), jnp.float32),
        ],
    )(x, lns, lnb, rw, rb)


def _pos_body(meta_ref, sidx_ref, slotcat_ref, keep_ref):
    e0 = meta_ref[:, 0:1].astype(jnp.int32)
    e1 = meta_ref[:, 1:2].astype(jnp.int32)
    iota = lax.broadcasted_iota(jnp.int32, (T, E), 1)
    oh0 = (e0 == iota).astype(jnp.int32)
    oh1 = (e1 == iota).astype(jnp.int32)
    both = oh0 + oh1
    s = both
    sh = 1
    while sh < T:
        top = jnp.zeros((sh, E), jnp.int32)
        s = s + jnp.concatenate([top, s[:-sh, :]], axis=0)
        sh *= 2
    s = s - both
    pos0 = jnp.sum(s * oh0, axis=1, keepdims=True)
    pos1 = jnp.sum(s * oh1, axis=1, keepdims=True)
    keep0 = pos0 < C
    keep1 = pos1 < C
    slot0 = e0 * C + jnp.minimum(pos0, C - 1)
    slot1 = e1 * C + jnp.minimum(pos1, C - 1)
    sidx_ref[...] = jnp.concatenate(
        [jnp.where(keep0, slot0, NSLOT), jnp.where(keep1, slot1, NSLOT)], axis=1)
    slotcat_ref[...] = jnp.concatenate([slot0, slot1], axis=0)
    keep_ref[...] = jnp.concatenate(
        [keep0.astype(jnp.float32), keep1.astype(jnp.float32)], axis=1)


def _pos(meta):
    return pl.pallas_call(
        _pos_body,
        out_shape=[
            jax.ShapeDtypeStruct((T, K), jnp.int32),
            jax.ShapeDtypeStruct((NPAIR, 1), jnp.int32),
            jax.ShapeDtypeStruct((T, K), jnp.float32),
        ],
    )(meta)


@functools.cache
def _sc_mesh():
    return plsc.VectorSubcoreMesh(core_axis_name="c", subcore_axis_name="s")


PW = D // 2


@functools.cache
def _make_dispatch(chunk):
    bw = NPAIR // NW
    nch = bw // chunk

    @functools.partial(
        pl.kernel,
        mesh=_sc_mesh(),
        out_type=jax.ShapeDtypeStruct((NSLOT + 8, PW), jnp.float32),
        scratch_types=[
            pltpu.VMEM((nch, chunk), jnp.int32),
            pltpu.VMEM((nch, chunk), jnp.int32),
            pltpu.VMEM((chunk, PW), jnp.float32),
            pltpu.SemaphoreType.DMA,
        ],
    )
    def k(h_hbm, tsrc_hbm, sidx_hbm, buf_hbm, tidx_v, sidx_v, rows_v, sem):
        wid = lax.axis_index("s") * 2 + lax.axis_index("c")
        pltpu.sync_copy(tsrc_hbm.at[wid], tidx_v)
        pltpu.sync_copy(sidx_hbm.at[wid], sidx_v)

        def body(c, carry):
            pltpu.async_copy(h_hbm.at[tidx_v.at[c]], rows_v, sem).wait()
            pltpu.async_copy(rows_v, buf_hbm.at[sidx_v.at[c]], sem).wait()
            return carry

        lax.fori_loop(0, nch, body, 0)

    return k


@functools.cache
def _make_gather(nrows_out, chunk):
    bw = nrows_out // NW
    nch = bw // chunk

    @functools.partial(
        pl.kernel,
        mesh=_sc_mesh(),
        out_type=jax.ShapeDtypeStruct((nrows_out, PW), jnp.float32),
        scratch_types=[
            pltpu.VMEM((nch, chunk), jnp.int32),
            pltpu.VMEM((chunk, PW), jnp.float32),
            pltpu.SemaphoreType.DMA,
        ],
    )
    def k(table_hbm, idx_hbm, out_hbm, idx_v, rows_v, sem):
        wid = lax.axis_index("s") * 2 + lax.axis_index("c")
        base = wid * bw
        pltpu.sync_copy(idx_hbm.at[wid], idx_v)

        def body(c, carry):
            off = pl.multiple_of(base + c * chunk, 8)
            pltpu.async_copy(table_hbm.at[idx_v.at[c]], rows_v, sem).wait()
            pltpu.sync_copy(rows_v, out_hbm.at[pl.ds(off, chunk)])
            return carry

        lax.fori_loop(0, nch, body, 0)

    return k


def _experts_body(buf_ref, w1_ref, b1_ref, w2_ref, b2_ref, out_ref):
    xw = buf_ref[...]
    a = (jnp.dot(_unpack_lo(xw), w1_ref[0, :D // 2],
                 preferred_element_type=jnp.float32)
         + jnp.dot(_unpack_hi(xw), w1_ref[0, D // 2:],
                   preferred_element_type=jnp.float32)
         + b1_ref[0])
    a = _gelu(a)
    res = jnp.dot(a, w2_ref[0], preferred_element_type=jnp.float32) + b2_ref[0]
    out_ref[...] = _pack_words(res[:, :D // 2], res[:, D // 2:])


def _experts(buf, w1, b1, w2, b2):
    return pl.pallas_call(
        _experts_body,
        grid=(E,),
        in_specs=[
            pl.BlockSpec((C, D // 2), lambda e: (e, 0)),
            pl.BlockSpec((1, D, H), lambda e: (e, 0, 0)),
            pl.BlockSpec((1, 1, H), lambda e: (e, 0, 0)),
            pl.BlockSpec((1, H, D), lambda e: (e, 0, 0)),
            pl.BlockSpec((1, 1, D), lambda e: (e, 0, 0)),
        ],
        out_specs=pl.BlockSpec((C, D // 2), lambda e: (e, 0)),
        out_shape=jax.ShapeDtypeStruct((NSLOT, D // 2), jnp.float32),
    )(buf, w1, b1, w2, b2)


def _final_body(x_ref, h_ref, ta_ref, tb_ref, meta_ref, keep_ref,
                w1_ref, b1_ref, w2_ref, b2_ref, out_ref):
    h = h_ref[...]
    a = _gelu(jnp.dot(h, w1_ref[...], preferred_element_type=jnp.float32) + b1_ref[...])
    sh = x_ref[...] + jnp.dot(a, w2_ref[...], preferred_element_type=jnp.float32) + b2_ref[...]
    gk0 = meta_ref[:, 2:3] * keep_ref[:, 0:1]
    gk1 = meta_ref[:, 3:4] * keep_ref[:, 1:2]
    ta = ta_ref[...]
    tb = tb_ref[...]
    out_ref[:, :D // 2] = sh[:, :D // 2] + ALPHA * (
        gk0 * _unpack_lo(ta) + gk1 * _unpack_lo(tb))
    out_ref[:, D // 2:] = sh[:, D // 2:] + ALPHA * (
        gk0 * _unpack_hi(ta) + gk1 * _unpack_hi(tb))


def _final(x, h, tokrows, meta, keep2, w1, b1, w2, b2):
    return pl.pallas_call(
        _final_body,
        grid=(T // RT,),
        in_specs=[
            pl.BlockSpec((RT, D), lambda i: (i, 0)),
            pl.BlockSpec((RT, D), lambda i: (i, 0)),
            pl.BlockSpec((RT, D // 2), lambda i: (i, 0)),
            pl.BlockSpec((RT, D // 2), lambda i: (i + T // RT, 0)),
            pl.BlockSpec((RT, 4), lambda i: (i, 0)),
            pl.BlockSpec((RT, K), lambda i: (i, 0)),
            pl.BlockSpec((D, SHARED_H), lambda i: (0, 0)),
            pl.BlockSpec((1, SHARED_H), lambda i: (0, 0)),
            pl.BlockSpec((SHARED_H, D), lambda i: (0, 0)),
            pl.BlockSpec((1, D), lambda i: (0, 0)),
        ],
        out_specs=pl.BlockSpec((RT, D), lambda i: (i, 0)),
        out_shape=jax.ShapeDtypeStruct((T, D), jnp.float32),
    )(x, h, tokrows, tokrows, meta, keep2, w1, b1, w2, b2)


def kernel(hidden_states, ln_scale, ln_bias, shared_W1, shared_b1, shared_W2,
           shared_b2, router_W, router_b, expert_W1, expert_b1, expert_W2,
           expert_b2):
    x = hidden_states.reshape(T, D)
    h, hb, meta = _lnrt(x, ln_scale.reshape(1, D), ln_bias.reshape(1, D),
                        router_W, router_b.reshape(1, E))
    sidx2, slotcat, keep2 = _pos(meta)
    chunk = 64
    tsrc = jnp.repeat(jnp.arange(T, dtype=jnp.int32), K).reshape(NW, -1, chunk)
    buf = _make_dispatch(chunk)(hb, tsrc, sidx2.reshape(NW, -1, chunk))
    eo = _experts(buf, expert_W1, expert_b1.reshape(E, 1, H),
                  expert_W2, expert_b2.reshape(E, 1, D))
    tokrows = _make_gather(NPAIR, chunk)(eo, slotcat.reshape(NW, -1, chunk))
    out = _final(x, h, tokrows, meta, keep2, shared_W1,
                 shared_b1.reshape(1, SHARED_H), shared_W2,
                 shared_b2.reshape(1, D))
    return out.reshape(1, T, D)


# speedup vs baseline: 3.3222x; 1.0081x over previous
import functools

import jax
import jax.numpy as jnp
from jax import lax
from jax.experimental import pallas as pl
from jax.experimental.pallas import tpu as pltpu
from jax.experimental.pallas import tpu_sc as plsc

T = 2048
D = 1024
E = 64
K = 2
H = 512
C = 96
SHARED_H = 1024
ALPHA = 0.5
EPS = 1e-05
NPAIR = T * K
NSLOT = E * C
RT = 256
NW = 32


def _gelu(x):
    return 0.5 * x * (1.0 + lax.erf(x * 0.7071067811865476))


def _pack_words(lo, hi):
    lo16 = lax.bitcast_convert_type(lo.astype(jnp.bfloat16),
                                    jnp.uint16).astype(jnp.uint32)
    hi16 = lax.bitcast_convert_type(hi.astype(jnp.bfloat16),
                                    jnp.uint16).astype(jnp.uint32)
    return lax.bitcast_convert_type(
        jnp.left_shift(hi16, jnp.uint32(16)) | lo16, jnp.float32)


def _unpack_lo(w):
    u = lax.bitcast_convert_type(w, jnp.uint32)
    return lax.bitcast_convert_type(
        jnp.left_shift(u, jnp.uint32(16)), jnp.float32)


def _unpack_hi(w):
    u = lax.bitcast_convert_type(w, jnp.uint32)
    return lax.bitcast_convert_type(u & jnp.uint32(0xFFFF0000), jnp.float32)


def _ln(x, lns, lnb):
    mu = jnp.mean(x, axis=1, keepdims=True)
    xc = x - mu
    var = jnp.mean(xc * xc, axis=1, keepdims=True)
    return xc * lax.rsqrt(var + EPS) * lns + lnb


def _lnrt_body(x_ref, lns_ref, lnb_ref, rw_ref, rb_ref, hb_ref, meta_ref):
    h = _ln(x_ref[...], lns_ref[...], lnb_ref[...])
    hb_ref[...] = _pack_words(h[:, :D // 2], h[:, D // 2:])
    logits = jnp.dot(h, rw_ref[...], preferred_element_type=jnp.float32) + rb_ref[...]
    col = lax.broadcasted_iota(jnp.int32, logits.shape, 1)
    v0 = jnp.max(logits, axis=1, keepdims=True)
    i0 = jnp.min(jnp.where(logits == v0, col, E), axis=1, keepdims=True)
    l2 = jnp.where(col == i0, -jnp.inf, logits)
    v1 = jnp.max(l2, axis=1, keepdims=True)
    i1 = jnp.min(jnp.where(l2 == v1, col, E), axis=1, keepdims=True)
    e1 = jnp.exp(v1 - v0)
    g0 = 1.0 / (1.0 + e1)
    g1 = e1 * g0
    meta_ref[...] = jnp.concatenate(
        [i0.astype(jnp.float32), i1.astype(jnp.float32), g0, g1], axis=1)


def _lnrt(x, lns, lnb, rw, rb):
    return pl.pallas_call(
        _lnrt_body,
        grid=(T // RT,),
        in_specs=[
            pl.BlockSpec((RT, D), lambda i: (i, 0)),
            pl.BlockSpec((1, D), lambda i: (0, 0)),
            pl.BlockSpec((1, D), lambda i: (0, 0)),
            pl.BlockSpec((D, E), lambda i: (0, 0)),
            pl.BlockSpec((1, E), lambda i: (0, 0)),
        ],
        out_specs=[
            pl.BlockSpec((RT, D // 2), lambda i: (i, 0)),
            pl.BlockSpec((RT, 4), lambda i: (i, 0)),
        ],
        out_shape=[
            jax.ShapeDtypeStruct((T, D // 2), jnp.float32),
            jax.ShapeDtypeStruct((T, 4), jnp.float32),
        ],
    )(x, lns, lnb, rw, rb)


def _pos_body(meta_ref, sidx_ref, slotcat_ref, keep_ref):
    e0 = meta_ref[:, 0:1].astype(jnp.int32)
    e1 = meta_ref[:, 1:2].astype(jnp.int32)
    iota = lax.broadcasted_iota(jnp.int32, (T, E), 1)
    oh0 = (e0 == iota).astype(jnp.int32)
    oh1 = (e1 == iota).astype(jnp.int32)
    both = oh0 + oh1
    s = both
    sh = 1
    while sh < T:
        top = jnp.zeros((sh, E), jnp.int32)
        s = s + jnp.concatenate([top, s[:-sh, :]], axis=0)
        sh *= 2
    s = s - both
    pos0 = jnp.sum(s * oh0, axis=1, keepdims=True)
    pos1 = jnp.sum(s * oh1, axis=1, keepdims=True)
    keep0 = pos0 < C
    keep1 = pos1 < C
    slot0 = e0 * C + jnp.minimum(pos0, C - 1)
    slot1 = e1 * C + jnp.minimum(pos1, C - 1)
    sidx_ref[...] = jnp.concatenate(
        [jnp.where(keep0, slot0, NSLOT), jnp.where(keep1, slot1, NSLOT)], axis=1)
    slotcat_ref[...] = jnp.concatenate([slot0, slot1], axis=0)
    keep_ref[...] = jnp.concatenate(
        [keep0.astype(jnp.float32), keep1.astype(jnp.float32)], axis=1)


def _pos(meta):
    return pl.pallas_call(
        _pos_body,
        out_shape=[
            jax.ShapeDtypeStruct((T, K), jnp.int32),
            jax.ShapeDtypeStruct((NPAIR, 1), jnp.int32),
            jax.ShapeDtypeStruct((T, K), jnp.float32),
        ],
    )(meta)


@functools.cache
def _sc_mesh():
    return plsc.VectorSubcoreMesh(core_axis_name="c", subcore_axis_name="s")


PW = D // 2


@functools.cache
def _make_dispatch(chunk):
    bw = NPAIR // NW
    nch = bw // chunk

    @functools.partial(
        pl.kernel,
        mesh=_sc_mesh(),
        out_type=jax.ShapeDtypeStruct((NSLOT + 8, PW), jnp.float32),
        scratch_types=[
            pltpu.VMEM((nch, chunk), jnp.int32),
            pltpu.VMEM((nch, chunk), jnp.int32),
            pltpu.VMEM((chunk, PW), jnp.float32),
            pltpu.SemaphoreType.DMA,
        ],
    )
    def k(h_hbm, tsrc_hbm, sidx_hbm, buf_hbm, tidx_v, sidx_v, rows_v, sem):
        wid = lax.axis_index("s") * 2 + lax.axis_index("c")
        pltpu.sync_copy(tsrc_hbm.at[wid], tidx_v)
        pltpu.sync_copy(sidx_hbm.at[wid], sidx_v)

        def body(c, carry):
            pltpu.async_copy(h_hbm.at[tidx_v.at[c]], rows_v, sem).wait()
            pltpu.async_copy(rows_v, buf_hbm.at[sidx_v.at[c]], sem).wait()
            return carry

        lax.fori_loop(0, nch, body, 0)

    return k


@functools.cache
def _make_gather(nrows_out, chunk):
    bw = nrows_out // NW
    nch = bw // chunk

    @functools.partial(
        pl.kernel,
        mesh=_sc_mesh(),
        out_type=jax.ShapeDtypeStruct((nrows_out, PW), jnp.float32),
        scratch_types=[
            pltpu.VMEM((nch, chunk), jnp.int32),
            pltpu.VMEM((chunk, PW), jnp.float32),
            pltpu.SemaphoreType.DMA,
        ],
    )
    def k(table_hbm, idx_hbm, out_hbm, idx_v, rows_v, sem):
        wid = lax.axis_index("s") * 2 + lax.axis_index("c")
        base = wid * bw
        pltpu.sync_copy(idx_hbm.at[wid], idx_v)

        def body(c, carry):
            off = pl.multiple_of(base + c * chunk, 8)
            pltpu.async_copy(table_hbm.at[idx_v.at[c]], rows_v, sem).wait()
            pltpu.sync_copy(rows_v, out_hbm.at[pl.ds(off, chunk)])
            return carry

        lax.fori_loop(0, nch, body, 0)

    return k


def _experts_body(buf_ref, w1_ref, b1_ref, w2_ref, b2_ref, out_ref):
    xw = buf_ref[...]
    a = (jnp.dot(_unpack_lo(xw), w1_ref[0, :D // 2],
                 preferred_element_type=jnp.float32)
         + jnp.dot(_unpack_hi(xw), w1_ref[0, D // 2:],
                   preferred_element_type=jnp.float32)
         + b1_ref[0])
    a = _gelu(a)
    res = jnp.dot(a, w2_ref[0], preferred_element_type=jnp.float32) + b2_ref[0]
    out_ref[...] = _pack_words(res[:, :D // 2], res[:, D // 2:])


def _experts(buf, w1, b1, w2, b2):
    return pl.pallas_call(
        _experts_body,
        grid=(E,),
        in_specs=[
            pl.BlockSpec((C, D // 2), lambda e: (e, 0)),
            pl.BlockSpec((1, D, H), lambda e: (e, 0, 0)),
            pl.BlockSpec((1, 1, H), lambda e: (e, 0, 0)),
            pl.BlockSpec((1, H, D), lambda e: (e, 0, 0)),
            pl.BlockSpec((1, 1, D), lambda e: (e, 0, 0)),
        ],
        out_specs=pl.BlockSpec((C, D // 2), lambda e: (e, 0)),
        out_shape=jax.ShapeDtypeStruct((NSLOT, D // 2), jnp.float32),
    )(buf, w1, b1, w2, b2)


def _final_body(x_ref, lns_ref, lnb_ref, ta_ref, tb_ref, meta_ref, keep_ref,
                w1_ref, b1_ref, w2_ref, b2_ref, out_ref):
    h = _ln(x_ref[...], lns_ref[...], lnb_ref[...])
    a = _gelu(jnp.dot(h, w1_ref[...], preferred_element_type=jnp.float32) + b1_ref[...])
    sh = x_ref[...] + jnp.dot(a, w2_ref[...], preferred_element_type=jnp.float32) + b2_ref[...]
    gk0 = meta_ref[:, 2:3] * keep_ref[:, 0:1]
    gk1 = meta_ref[:, 3:4] * keep_ref[:, 1:2]
    ta = ta_ref[...]
    tb = tb_ref[...]
    out_ref[:, :D // 2] = sh[:, :D // 2] + ALPHA * (
        gk0 * _unpack_lo(ta) + gk1 * _unpack_lo(tb))
    out_ref[:, D // 2:] = sh[:, D // 2:] + ALPHA * (
        gk0 * _unpack_hi(ta) + gk1 * _unpack_hi(tb))


def _final(x, lns, lnb, tokrows, meta, keep2, w1, b1, w2, b2):
    return pl.pallas_call(
        _final_body,
        grid=(T // RT,),
        in_specs=[
            pl.BlockSpec((RT, D), lambda i: (i, 0)),
            pl.BlockSpec((1, D), lambda i: (0, 0)),
            pl.BlockSpec((1, D), lambda i: (0, 0)),
            pl.BlockSpec((RT, D // 2), lambda i: (i, 0)),
            pl.BlockSpec((RT, D // 2), lambda i: (i + T // RT, 0)),
            pl.BlockSpec((RT, 4), lambda i: (i, 0)),
            pl.BlockSpec((RT, K), lambda i: (i, 0)),
            pl.BlockSpec((D, SHARED_H), lambda i: (0, 0)),
            pl.BlockSpec((1, SHARED_H), lambda i: (0, 0)),
            pl.BlockSpec((SHARED_H, D), lambda i: (0, 0)),
            pl.BlockSpec((1, D), lambda i: (0, 0)),
        ],
        out_specs=pl.BlockSpec((RT, D), lambda i: (i, 0)),
        out_shape=jax.ShapeDtypeStruct((T, D), jnp.float32),
    )(x, lns, lnb, tokrows, tokrows, meta, keep2, w1, b1, w2, b2)


def kernel(hidden_states, ln_scale, ln_bias, shared_W1, shared_b1, shared_W2,
           shared_b2, router_W, router_b, expert_W1, expert_b1, expert_W2,
           expert_b2):
    x = hidden_states.reshape(T, D)
    lns = ln_scale.reshape(1, D)
    lnb = ln_bias.reshape(1, D)
    hb, meta = _lnrt(x, lns, lnb, router_W, router_b.reshape(1, E))
    sidx2, slotcat, keep2 = _pos(meta)
    chunk = 64
    tsrc = jnp.repeat(jnp.arange(T, dtype=jnp.int32), K).reshape(NW, -1, chunk)
    buf = _make_dispatch(chunk)(hb, tsrc, sidx2.reshape(NW, -1, chunk))
    eo = _experts(buf, expert_W1, expert_b1.reshape(E, 1, H),
                  expert_W2, expert_b2.reshape(E, 1, D))
    tokrows = _make_gather(NPAIR, chunk)(eo, slotcat.reshape(NW, -1, chunk))
    out = _final(x, lns, lnb, tokrows, meta, keep2, shared_W1,
                 shared_b1.reshape(1, SHARED_H), shared_W2,
                 shared_b2.reshape(1, D))
    return out.reshape(1, T, D)


# speedup vs baseline: 3.3824x; 1.0181x over previous
import functools

import jax
import jax.numpy as jnp
from jax import lax
from jax.experimental import pallas as pl
from jax.experimental.pallas import tpu as pltpu
from jax.experimental.pallas import tpu_sc as plsc

T = 2048
D = 1024
E = 64
K = 2
H = 512
C = 96
SHARED_H = 1024
ALPHA = 0.5
EPS = 1e-05
NPAIR = T * K
NSLOT = E * C
RT = 256
NW = 32


def _gelu(x):
    return 0.5 * x * (1.0 + lax.erf(x * 0.7071067811865476))


def _pack_words(lo, hi):
    lo16 = lax.bitcast_convert_type(lo.astype(jnp.bfloat16),
                                    jnp.uint16).astype(jnp.uint32)
    hi16 = lax.bitcast_convert_type(hi.astype(jnp.bfloat16),
                                    jnp.uint16).astype(jnp.uint32)
    return lax.bitcast_convert_type(
        jnp.left_shift(hi16, jnp.uint32(16)) | lo16, jnp.float32)


def _unpack_lo(w):
    u = lax.bitcast_convert_type(w, jnp.uint32)
    return lax.bitcast_convert_type(
        jnp.left_shift(u, jnp.uint32(16)), jnp.float32)


def _unpack_hi(w):
    u = lax.bitcast_convert_type(w, jnp.uint32)
    return lax.bitcast_convert_type(u & jnp.uint32(0xFFFF0000), jnp.float32)


def _ln(x, lns, lnb):
    mu = jnp.mean(x, axis=1, keepdims=True)
    xc = x - mu
    var = jnp.mean(xc * xc, axis=1, keepdims=True)
    return xc * lax.rsqrt(var + EPS) * lns + lnb


def _lnrt_body(x_ref, lns_ref, lnb_ref, rw_ref, rb_ref, hb_ref, meta_ref):
    h = _ln(x_ref[...], lns_ref[...], lnb_ref[...])
    hb_ref[...] = _pack_words(h[:, :D // 2], h[:, D // 2:])
    logits = jnp.dot(h, rw_ref[...], preferred_element_type=jnp.float32) + rb_ref[...]
    col = lax.broadcasted_iota(jnp.int32, logits.shape, 1)
    v0 = jnp.max(logits, axis=1, keepdims=True)
    i0 = jnp.min(jnp.where(logits == v0, col, E), axis=1, keepdims=True)
    l2 = jnp.where(col == i0, -jnp.inf, logits)
    v1 = jnp.max(l2, axis=1, keepdims=True)
    i1 = jnp.min(jnp.where(l2 == v1, col, E), axis=1, keepdims=True)
    e1 = jnp.exp(v1 - v0)
    g0 = 1.0 / (1.0 + e1)
    g1 = e1 * g0
    meta_ref[...] = jnp.concatenate(
        [i0.astype(jnp.float32), i1.astype(jnp.float32), g0, g1], axis=1)


def _lnrt(x, lns, lnb, rw, rb):
    return pl.pallas_call(
        _lnrt_body,
        grid=(T // RT,),
        in_specs=[
            pl.BlockSpec((RT, D), lambda i: (i, 0)),
            pl.BlockSpec((1, D), lambda i: (0, 0)),
            pl.BlockSpec((1, D), lambda i: (0, 0)),
            pl.BlockSpec((D, E), lambda i: (0, 0)),
            pl.BlockSpec((1, E), lambda i: (0, 0)),
        ],
        out_specs=[
            pl.BlockSpec((RT, D // 2), lambda i: (i, 0)),
            pl.BlockSpec((RT, 4), lambda i: (i, 0)),
        ],
        out_shape=[
            jax.ShapeDtypeStruct((T, D // 2), jnp.float32),
            jax.ShapeDtypeStruct((T, 4), jnp.float32),
        ],
    )(x, lns, lnb, rw, rb)


def _pos_body(meta_ref, sidx_ref, slotcat_ref, keep_ref):
    e0 = meta_ref[:, 0:1].astype(jnp.int32)
    e1 = meta_ref[:, 1:2].astype(jnp.int32)
    iota = lax.broadcasted_iota(jnp.int32, (T, E), 1)
    oh0 = (e0 == iota).astype(jnp.int32)
    oh1 = (e1 == iota).astype(jnp.int32)
    both = oh0 + oh1
    s = both
    sh = 1
    while sh < T:
        top = jnp.zeros((sh, E), jnp.int32)
        s = s + jnp.concatenate([top, s[:-sh, :]], axis=0)
        sh *= 2
    s = s - both
    pos0 = jnp.sum(s * oh0, axis=1, keepdims=True)
    pos1 = jnp.sum(s * oh1, axis=1, keepdims=True)
    keep0 = pos0 < C
    keep1 = pos1 < C
    slot0 = e0 * C + jnp.minimum(pos0, C - 1)
    slot1 = e1 * C + jnp.minimum(pos1, C - 1)
    sidx_ref[...] = jnp.concatenate(
        [jnp.where(keep0, slot0, NSLOT), jnp.where(keep1, slot1, NSLOT)], axis=1)
    slotcat_ref[...] = jnp.concatenate([slot0, slot1], axis=0)
    keep_ref[...] = jnp.concatenate(
        [keep0.astype(jnp.float32), keep1.astype(jnp.float32)], axis=1)


def _pos(meta):
    return pl.pallas_call(
        _pos_body,
        out_shape=[
            jax.ShapeDtypeStruct((T, K), jnp.int32),
            jax.ShapeDtypeStruct((NPAIR, 1), jnp.int32),
            jax.ShapeDtypeStruct((T, K), jnp.float32),
        ],
    )(meta)


@functools.cache
def _sc_mesh():
    return plsc.VectorSubcoreMesh(core_axis_name="c", subcore_axis_name="s")


PW = D // 2


@functools.cache
def _make_dispatch(chunk):
    bw = NPAIR // NW
    nch = bw // chunk

    @functools.partial(
        pl.kernel,
        mesh=_sc_mesh(),
        out_type=jax.ShapeDtypeStruct((NSLOT + 8, PW), jnp.float32),
        scratch_types=[
            pltpu.VMEM((nch, chunk), jnp.int32),
            pltpu.VMEM((nch, chunk), jnp.int32),
            pltpu.VMEM((chunk, PW), jnp.float32),
            pltpu.SemaphoreType.DMA,
        ],
    )
    def k(h_hbm, tsrc_hbm, sidx_hbm, buf_hbm, tidx_v, sidx_v, rows_v, sem):
        wid = lax.axis_index("s") * 2 + lax.axis_index("c")
        pltpu.sync_copy(tsrc_hbm.at[wid], tidx_v)
        pltpu.sync_copy(sidx_hbm.at[wid], sidx_v)

        def body(c, carry):
            pltpu.async_copy(h_hbm.at[tidx_v.at[c]], rows_v, sem).wait()
            pltpu.async_copy(rows_v, buf_hbm.at[sidx_v.at[c]], sem).wait()
            return carry

        lax.fori_loop(0, nch, body, 0)

    return k


@functools.cache
def _make_gather(nrows_out, chunk):
    bw = nrows_out // NW
    nch = bw // chunk

    @functools.partial(
        pl.kernel,
        mesh=_sc_mesh(),
        out_type=jax.ShapeDtypeStruct((nrows_out, PW), jnp.float32),
        scratch_types=[
            pltpu.VMEM((nch, chunk), jnp.int32),
            pltpu.VMEM((chunk, PW), jnp.float32),
            pltpu.SemaphoreType.DMA,
        ],
    )
    def k(table_hbm, idx_hbm, out_hbm, idx_v, rows_v, sem):
        wid = lax.axis_index("s") * 2 + lax.axis_index("c")
        base = wid * bw
        pltpu.sync_copy(idx_hbm.at[wid], idx_v)

        def body(c, carry):
            off = pl.multiple_of(base + c * chunk, 8)
            pltpu.async_copy(table_hbm.at[idx_v.at[c]], rows_v, sem).wait()
            pltpu.sync_copy(rows_v, out_hbm.at[pl.ds(off, chunk)])
            return carry

        lax.fori_loop(0, nch, body, 0)

    return k


def _experts_body(buf_ref, w1_ref, b1_ref, w2_ref, b2_ref, out_ref):
    xw = buf_ref[...]
    a = (jnp.dot(_unpack_lo(xw), w1_ref[0, :D // 2],
                 preferred_element_type=jnp.float32)
         + jnp.dot(_unpack_hi(xw), w1_ref[0, D // 2:],
                   preferred_element_type=jnp.float32)
         + b1_ref[0])
    a = _gelu(a)
    res = jnp.dot(a, w2_ref[0], preferred_element_type=jnp.float32) + b2_ref[0]
    out_ref[...] = _pack_words(res[:, :D // 2], res[:, D // 2:])


def _experts(buf, w1, b1, w2, b2):
    return pl.pallas_call(
        _experts_body,
        grid=(E,),
        in_specs=[
            pl.BlockSpec((C, D // 2), lambda e: (e, 0)),
            pl.BlockSpec((1, D, H), lambda e: (e, 0, 0)),
            pl.BlockSpec((1, 1, H), lambda e: (e, 0, 0)),
            pl.BlockSpec((1, H, D), lambda e: (e, 0, 0)),
            pl.BlockSpec((1, 1, D), lambda e: (e, 0, 0)),
        ],
        out_specs=pl.BlockSpec((C, D // 2), lambda e: (e, 0)),
        out_shape=jax.ShapeDtypeStruct((NSLOT, D // 2), jnp.float32),
    )(buf, w1, b1, w2, b2)


def _final_body(x_ref, lns_ref, lnb_ref, ta_ref, tb_ref, meta_ref, keep_ref,
                w1_ref, b1_ref, w2_ref, b2_ref, out_ref):
    h = _ln(x_ref[...], lns_ref[...], lnb_ref[...])
    a = _gelu(jnp.dot(h, w1_ref[...], preferred_element_type=jnp.float32) + b1_ref[...])
    sh = x_ref[...] + jnp.dot(a, w2_ref[...], preferred_element_type=jnp.float32) + b2_ref[...]
    gk0 = meta_ref[:, 2:3] * keep_ref[:, 0:1]
    gk1 = meta_ref[:, 3:4] * keep_ref[:, 1:2]
    ta = ta_ref[...]
    tb = tb_ref[...]
    out_ref[:, :D // 2] = sh[:, :D // 2] + ALPHA * (
        gk0 * _unpack_lo(ta) + gk1 * _unpack_lo(tb))
    out_ref[:, D // 2:] = sh[:, D // 2:] + ALPHA * (
        gk0 * _unpack_hi(ta) + gk1 * _unpack_hi(tb))


def _final(x, lns, lnb, tokrows, meta, keep2, w1, b1, w2, b2):
    return pl.pallas_call(
        _final_body,
        grid=(T // RT,),
        in_specs=[
            pl.BlockSpec((RT, D), lambda i: (i, 0)),
            pl.BlockSpec((1, D), lambda i: (0, 0)),
            pl.BlockSpec((1, D), lambda i: (0, 0)),
            pl.BlockSpec((RT, D // 2), lambda i: (i, 0)),
            pl.BlockSpec((RT, D // 2), lambda i: (i + T // RT, 0)),
            pl.BlockSpec((RT, 4), lambda i: (i, 0)),
            pl.BlockSpec((RT, K), lambda i: (i, 0)),
            pl.BlockSpec((D, SHARED_H), lambda i: (0, 0)),
            pl.BlockSpec((1, SHARED_H), lambda i: (0, 0)),
            pl.BlockSpec((SHARED_H, D), lambda i: (0, 0)),
            pl.BlockSpec((1, D), lambda i: (0, 0)),
        ],
        out_specs=pl.BlockSpec((RT, D), lambda i: (i, 0)),
        out_shape=jax.ShapeDtypeStruct((T, D), jnp.float32),
    )(x, lns, lnb, tokrows, tokrows, meta, keep2, w1, b1, w2, b2)


def kernel(hidden_states, ln_scale, ln_bias, shared_W1, shared_b1, shared_W2,
           shared_b2, router_W, router_b, expert_W1, expert_b1, expert_W2,
           expert_b2):
    x = hidden_states.reshape(T, D)
    lns = ln_scale.reshape(1, D)
    lnb = ln_bias.reshape(1, D)
    hb, meta = _lnrt(x, lns, lnb, router_W, router_b.reshape(1, E))
    sidx2, slotcat, keep2 = _pos(meta)
    chunk = 128
    tsrc = jnp.repeat(jnp.arange(T, dtype=jnp.int32), K).reshape(NW, -1, chunk)
    buf = _make_dispatch(chunk)(hb, tsrc, sidx2.reshape(NW, -1, chunk))
    eo = _experts(buf, expert_W1, expert_b1.reshape(E, 1, H),
                  expert_W2, expert_b2.reshape(E, 1, D))
    tokrows = _make_gather(NPAIR, chunk)(eo, slotcat.reshape(NW, -1, chunk))
    out = _final(x, lns, lnb, tokrows, meta, keep2, shared_W1,
                 shared_b1.reshape(1, SHARED_H), shared_W2,
                 shared_b2.reshape(1, D))
    return out.reshape(1, T, D)
